# baseline (speedup 1.0000x reference)
import functools

import jax
import jax.numpy as jnp
from jax import lax
from jax.experimental import pallas as pl
from jax.experimental.pallas import tpu as pltpu
from jax.experimental.pallas import tpu_sc as plsc

VOCAB = 1000000
EMBED_DIM = 64
BATCH = 4096
SEQ = 200

NC = 2
NS = 16
NW = NC * NS
RPW = BATCH // NW
SEQ_HI = SEQ // 8
NKV = EMBED_DIM // 16


def _encoder_kernel(pair_hbm, off_hbm, table_hbm, out_hbm,
                    idx_v, off_v, buf0, buf1, out_v, sem0, sem1):
    wid = lax.axis_index("s") * NC + lax.axis_index("c")

    inv = jnp.float32(1.0 / SEQ)
    bufs = (buf0, buf1)
    sems = (sem0, sem1)

    pltpu.sync_copy(pair_hbm.at[:, wid], idx_v)
    pltpu.sync_copy(off_hbm.at[:, wid], off_v)

    def zero_body(f, carry):
        z = jnp.zeros((16,), jnp.float32)
        for c in range(RPW // 16):
            out_v[f, pl.ds(16 * c, 16)] = z
        return carry

    lax.fori_loop(0, EMBED_DIM, zero_body, 0)

    def fire(j, slot):
        pltpu.async_copy(
            table_hbm.at[idx_v.at[j >> 3, j & 7]], bufs[slot], sems[slot])

    def wait(slot):
        pltpu.make_async_copy(
            table_hbm.at[idx_v.at[0, 0]], bufs[slot], sems[slot]).wait()

    iotas = [lax.iota(jnp.int32, 16) + i0 for i0 in range(0, RPW, 16)]

    def accum(j, slot):
        buf = bufs[slot]
        hi = j >> 3
        lo = j & 7
        for g in range(RPW // 16):
            off_vec = off_v[hi, lo, pl.ds(16 * g, 16)]
            rows16 = iotas[g]
            for f in range(EMBED_DIM):
                v = plsc.load_gather(buf, [rows16, off_vec + f])
                plsc.addupdate(out_v.at[f, pl.ds(16 * g, 16)], v)

    fire(0, 0)

    def outer(jj, carry):
        for s in range(2):
            j = 2 * jj + s

            @pl.when(j + 1 < SEQ)
            def _():
                fire(j + 1, 1 - s)

            wait(s)
            accum(j, s)
        return carry

    lax.fori_loop(0, SEQ // 2, outer, 0)

    def scale_body(f, carry):
        for c in range(RPW // 16):
            out_v[f, pl.ds(16 * c, 16)] = out_v[f, pl.ds(16 * c, 16)] * inv
        return carry

    lax.fori_loop(0, EMBED_DIM, scale_body, 0)
    pltpu.sync_copy(out_v, out_hbm.at[:, pl.ds(wid * RPW, RPW)])


def kernel(text_ids, table):
    ids = text_ids.astype(jnp.int32)
    def as4d(x):
        return x.T.reshape(SEQ_HI, 8, NW, RPW).transpose(0, 2, 1, 3)

    pair4d = as4d(ids >> 1)
    off4d = as4d((ids & 1) << 6)
    table2 = table.reshape(VOCAB // 2, 2 * EMBED_DIM)
    mesh = plsc.VectorSubcoreMesh(core_axis_name="c", subcore_axis_name="s")
    k = functools.partial(
        pl.kernel,
        mesh=mesh,
        out_type=jax.ShapeDtypeStruct((EMBED_DIM, BATCH), jnp.float32),
        scratch_types=[
            pltpu.VMEM((SEQ_HI, 8, RPW), jnp.int32),
            pltpu.VMEM((SEQ_HI, 8, RPW), jnp.int32),
            pltpu.VMEM((RPW, 2 * EMBED_DIM), jnp.float32),
            pltpu.VMEM((RPW, 2 * EMBED_DIM), jnp.float32),
            pltpu.VMEM((EMBED_DIM, RPW), jnp.float32),
            pltpu.SemaphoreType.DMA,
            pltpu.SemaphoreType.DMA,
        ],
        compiler_params=pltpu.CompilerParams(
            use_tc_tiling_on_sc=True, needs_layout_passes=False),
    )(_encoder_kernel)
    return k(pair4d, off4d, table2).T
